# baseline (speedup 1.0000x reference)
import functools

import jax
import jax.numpy as jnp
from jax import lax
from jax.experimental import pallas as pl
from jax.experimental.pallas import tpu as pltpu
from jax.experimental.pallas import tpu_sc as plsc

VOCAB = 3
EMBED = 512
LANES = 16
ROWS = 4 * 8192
NUM_CORES = 2
NUM_SUBCORES = 16
NW = NUM_CORES * NUM_SUBCORES
R_PER_W = ROWS // NW
CHUNK = 32
NBUF = 4
NCHUNK = R_PER_W // CHUNK

_mesh = plsc.VectorSubcoreMesh(core_axis_name="c", subcore_axis_name="s")


@functools.partial(
    pl.kernel,
    mesh=_mesh,
    out_type=jax.ShapeDtypeStruct((ROWS, EMBED), jnp.float32),
    scratch_types=[
        pltpu.VMEM((R_PER_W,), jnp.int32),
        pltpu.VMEM((VOCAB, EMBED), jnp.float32),
        pltpu.VMEM((NBUF, CHUNK, EMBED), jnp.float32),
        pltpu.VMEM_SHARED((NUM_SUBCORES, R_PER_W), jnp.int32),
        pltpu.SMEM((CHUNK,), jnp.int32),
        pltpu.SemaphoreType.DMA,
    ],
)
def _embed_sc(idx_hbm, w_hbm, out_hbm, idx_v, tbl_v, bufn, idx_sh,
              idx_sm, ssem):
    sid = lax.axis_index("s")
    wid = sid * NUM_CORES + lax.axis_index("c")
    base = wid * R_PER_W

    pltpu.sync_copy(w_hbm, tbl_v)
    pltpu.sync_copy(idx_hbm.at[pl.ds(base, R_PER_W)], idx_v)
    pltpu.sync_copy(idx_v, idx_sh.at[sid])

    bufs = tuple(bufn.at[b] for b in range(NBUF))

    def fill(c, buf):
        pltpu.sync_copy(idx_sh.at[sid, pl.ds(c * CHUNK, CHUNK)], idx_sm)

        @plsc.parallel_loop(0, CHUNK, 1, unroll=8)
        def row(i):
            r = idx_sm[i]
            for k in range(EMBED // LANES):
                buf[i, pl.ds(k * LANES, LANES)] = tbl_v[r, pl.ds(k * LANES, LANES)]

    def wait_one_scatter(b):
        pltpu.make_async_copy(
            bufs[b], out_hbm.at[pl.ds(base, CHUNK)], ssem
        ).wait()

    def outer(g, carry):
        for b in range(NBUF):
            c = g * NBUF + b

            @pl.when(c >= NBUF)
            def _():
                wait_one_scatter(b)

            fill(c, bufs[b])
            pltpu.async_copy(
                bufs[b], out_hbm.at[pl.ds(base + c * CHUNK, CHUNK)], ssem
            )
        return carry

    lax.fori_loop(0, NCHUNK // NBUF, outer, 0)
    for b in range(NBUF):
        wait_one_scatter(b)


def kernel(input, weight):
    idx = input.reshape(-1).astype(jnp.int32)
    out = _embed_sc(idx, weight)
    return out.reshape(input.shape + (EMBED,))


# speedup vs baseline: 1.2404x; 1.2404x over previous
import functools

import jax
import jax.numpy as jnp
from jax import lax
from jax.experimental import pallas as pl
from jax.experimental.pallas import tpu as pltpu
from jax.experimental.pallas import tpu_sc as plsc

VOCAB = 3
EMBED = 512
LANES = 16
ROWS = 4 * 8192
NUM_CORES = 2
NUM_SUBCORES = 16
NW = NUM_CORES * NUM_SUBCORES
R_PER_W = ROWS // NW
CHUNK = 64
NBUF = 3
NCHUNK = R_PER_W // CHUNK

_mesh = plsc.VectorSubcoreMesh(core_axis_name="c", subcore_axis_name="s")


@functools.partial(
    pl.kernel,
    mesh=_mesh,
    out_type=jax.ShapeDtypeStruct((ROWS, EMBED), jnp.float32),
    scratch_types=[
        pltpu.VMEM((R_PER_W,), jnp.int32),
        pltpu.VMEM((VOCAB, EMBED), jnp.float32),
        pltpu.VMEM((NBUF, CHUNK, EMBED), jnp.float32),
        pltpu.VMEM_SHARED((NUM_SUBCORES, R_PER_W), jnp.int32),
        pltpu.SMEM((CHUNK,), jnp.int32),
        pltpu.SemaphoreType.DMA((NBUF,)),
    ],
)
def _embed_sc(idx_hbm, w_hbm, out_hbm, idx_v, tbl_v, bufn, idx_sh,
              idx_sm, ssem):
    sid = lax.axis_index("s")
    wid = sid * NUM_CORES + lax.axis_index("c")
    base = wid * R_PER_W

    pltpu.sync_copy(w_hbm, tbl_v)
    pltpu.sync_copy(idx_hbm.at[pl.ds(base, R_PER_W)], idx_v)
    pltpu.sync_copy(idx_v, idx_sh.at[sid])

    bufs = tuple(bufn.at[b] for b in range(NBUF))

    def fill(c, buf):
        pltpu.sync_copy(idx_sh.at[sid, pl.ds(c * CHUNK, CHUNK)], idx_sm)

        @plsc.parallel_loop(0, CHUNK, 1, unroll=8)
        def row(i):
            r = idx_sm[i]
            for k in range(EMBED // LANES):
                buf[i, pl.ds(k * LANES, LANES)] = tbl_v[r, pl.ds(k * LANES, LANES)]

    def wait_scatter(b):
        pltpu.make_async_copy(
            bufs[b], out_hbm.at[pl.ds(base, CHUNK)], ssem.at[b]
        ).wait()

    def start_scatter(c, b):
        pltpu.async_copy(
            bufs[b], out_hbm.at[pl.ds(base + c * CHUNK, CHUNK)], ssem.at[b]
        )

    def outer(g, carry):
        for b in range(NBUF):
            c = g * NBUF + b

            @pl.when(c >= NBUF)
            def _():
                wait_scatter(b)

            fill(c, bufs[b])
            start_scatter(c, b)
        return carry

    nfull = (NCHUNK // NBUF) * NBUF
    lax.fori_loop(0, NCHUNK // NBUF, outer, 0)
    for c in range(nfull, NCHUNK):
        b = c - nfull
        wait_scatter(b)
        fill(c, bufs[b])
        start_scatter(c, b)
    for b in range(NBUF):
        wait_scatter(b)


def kernel(input, weight):
    idx = input.reshape(-1).astype(jnp.int32)
    out = _embed_sc(idx, weight)
    return out.reshape(input.shape + (EMBED,))


# speedup vs baseline: 1.3823x; 1.1144x over previous
import functools

import jax
import jax.numpy as jnp
from jax import lax
from jax.experimental import pallas as pl
from jax.experimental.pallas import tpu as pltpu
from jax.experimental.pallas import tpu_sc as plsc

VOCAB = 3
EMBED = 512
LANES = 16
ROWS = 4 * 8192
NUM_CORES = 2
NUM_SUBCORES = 16
NW = NUM_CORES * NUM_SUBCORES
R_PER_W = ROWS // NW
CHUNK = 64
NBUF = 2
NCHUNK = R_PER_W // CHUNK

_mesh = plsc.VectorSubcoreMesh(core_axis_name="c", subcore_axis_name="s")


@functools.partial(
    pl.kernel,
    mesh=_mesh,
    out_type=jax.ShapeDtypeStruct((ROWS, EMBED), jnp.float32),
    scratch_types=[
        pltpu.VMEM((R_PER_W,), jnp.int32),
        pltpu.VMEM((VOCAB, EMBED), jnp.float32),
        pltpu.VMEM((NBUF, CHUNK, EMBED), jnp.float32),
        pltpu.VMEM_SHARED((NUM_SUBCORES, R_PER_W), jnp.int32),
        pltpu.SMEM((CHUNK,), jnp.int32),
        pltpu.SemaphoreType.DMA((NBUF,)),
    ],
)
def _embed_sc(idx_hbm, w_hbm, out_hbm, idx_v, tbl_v, bufn, idx_sh,
              idx_sm, ssem):
    sid = lax.axis_index("s")
    wid = sid * NUM_CORES + lax.axis_index("c")
    base = wid * R_PER_W

    pltpu.sync_copy(w_hbm, tbl_v)
    pltpu.sync_copy(idx_hbm.at[pl.ds(base, R_PER_W)], idx_v)
    pltpu.sync_copy(idx_v, idx_sh.at[sid])

    bufs = tuple(bufn.at[b] for b in range(NBUF))

    def fill(c, buf):
        pltpu.sync_copy(idx_sh.at[sid, pl.ds(c * CHUNK, CHUNK)], idx_sm)

        @plsc.parallel_loop(0, CHUNK, 1, unroll=8)
        def row(i):
            r = idx_sm[i]
            for k in range(EMBED // LANES):
                buf[i, pl.ds(k * LANES, LANES)] = tbl_v[r, pl.ds(k * LANES, LANES)]

    def wait_scatter(b):
        pltpu.make_async_copy(
            bufs[b], out_hbm.at[pl.ds(base, CHUNK)], ssem.at[b]
        ).wait()

    def start_scatter(c, b):
        pltpu.async_copy(
            bufs[b], out_hbm.at[pl.ds(base + c * CHUNK, CHUNK)], ssem.at[b]
        )

    def outer(g, carry):
        for b in range(NBUF):
            c = g * NBUF + b

            @pl.when(c >= NBUF)
            def _():
                wait_scatter(b)

            fill(c, bufs[b])
            start_scatter(c, b)
        return carry

    nfull = (NCHUNK // NBUF) * NBUF
    lax.fori_loop(0, NCHUNK // NBUF, outer, 0)
    for c in range(nfull, NCHUNK):
        b = c - nfull
        wait_scatter(b)
        fill(c, bufs[b])
        start_scatter(c, b)
    for b in range(NBUF):
        wait_scatter(b)


def kernel(input, weight):
    idx = input.reshape(-1).astype(jnp.int32)
    out = _embed_sc(idx, weight)
    return out.reshape(input.shape + (EMBED,))
